# baseline (speedup 1.0000x reference)
import functools

import jax
import jax.numpy as jnp
from jax import lax
from jax.experimental import pallas as pl
from jax.experimental.pallas import tpu as pltpu
from jax.experimental.pallas import tpu_sc as plsc

N_IN = 2048
N_TOTAL = 8192
C = 128
K = 4


BQ = 1024
N_HALF = N_TOTAL // 2
_NB_HALF = N_HALF // BQ


def _topk_body(qt_ref, cf_ref, sh_ref, idx_ref):
    s = sh_ref[...]
    qx = qt_ref[0:1, :] - s[0:1, 0:1]
    qy = qt_ref[1:2, :] - s[0:1, 1:2]
    cf = cf_ref[...]
    dx = qx - cf[:, 0:1]
    dy = qy - cf[:, 1:2]
    d = dx * dx + dy * dy
    jj = jnp.broadcast_to(
        lax.broadcasted_iota(jnp.int32, (N_IN, 1), 0).astype(jnp.float32),
        (N_IN, BQ))
    rows = []
    for k in range(K):
        m = jnp.min(d, axis=0, keepdims=True)
        cand = jnp.where(d == m, jj, jnp.float32(N_IN))
        amin = jnp.min(cand, axis=0, keepdims=True)
        rows.append(amin)
        if k + 1 < K:
            d = jnp.where(jj == amin, jnp.float32(jnp.inf), d)
    idx_ref[0:K, :] = jnp.concatenate(rows, axis=0).astype(jnp.int32)


def _topk_valt_body(qt_ref, cf_ref, sh_ref, v_ref, idx_ref, valt_ref):
    _topk_body(qt_ref, cf_ref, sh_ref, idx_ref)
    valt_ref[...] = v_ref[...].T


def _make_topk(half, with_valt):
    qt_spec = pl.BlockSpec((2, BQ), lambda i: (0, i + half * _NB_HALF))
    common = [
        pl.BlockSpec((N_IN, 2), lambda i: (0, 0)),
        pl.BlockSpec((1, 2), lambda i: (0, 0)),
    ]
    idx_spec = pl.BlockSpec((2 * K, BQ), lambda i: (0, i))
    idx_shape = jax.ShapeDtypeStruct((2 * K, N_HALF), jnp.int32)
    if with_valt:
        return pl.pallas_call(
            _topk_valt_body,
            grid=(_NB_HALF,),
            in_specs=[qt_spec] + common + [
                pl.BlockSpec((C, N_IN // _NB_HALF), lambda i: (0, i))],
            out_specs=(idx_spec,
                       pl.BlockSpec((N_IN // _NB_HALF, C), lambda i: (i, 0))),
            out_shape=(idx_shape,
                       jax.ShapeDtypeStruct((N_IN, C), jnp.float32)),
        )
    return pl.pallas_call(
        _topk_body,
        grid=(_NB_HALF,),
        in_specs=[qt_spec] + common,
        out_specs=idx_spec,
        out_shape=idx_shape,
    )


_topk_a = _make_topk(0, True)
_topk_b = _make_topk(1, False)


_NC = 2
_NS = 16
_NW = _NC * _NS
_QPW = N_HALF // _NW


@functools.cache
def _make_sc_gather_mean():
    @functools.partial(
        pl.kernel,
        mesh=plsc.VectorSubcoreMesh(core_axis_name="c", subcore_axis_name="s"),
        out_type=jax.ShapeDtypeStruct((N_HALF, C), jnp.float32),
        scratch_types=[
            pltpu.VMEM((K, _QPW), jnp.int32),
            pltpu.VMEM((_QPW // 2, C), jnp.float32),
            pltpu.VMEM((_QPW // 2, C), jnp.float32),
            pltpu.VMEM((_QPW // 2, C), jnp.float32),
            pltpu.VMEM((_QPW // 2, C), jnp.float32),
            pltpu.VMEM((_QPW // 2, C), jnp.float32),
            pltpu.VMEM((_QPW // 2, C), jnp.float32),
            pltpu.VMEM((_QPW // 2, C), jnp.float32),
            pltpu.VMEM((_QPW // 2, C), jnp.float32),
            pltpu.VMEM((_QPW, C), jnp.float32),
            pltpu.SemaphoreType.DMA,
            pltpu.SemaphoreType.DMA,
            pltpu.SemaphoreType.DMA,
        ],
    )
    def _sc_gather_mean(valt_hbm, idxk_hbm, out_hbm, idx_v,
                        a0, a1, a2, a3, b0, b1, b2, b3,
                        out_v, semI, semA, semB):
        wid = lax.axis_index("s") * _NC + lax.axis_index("c")
        base_q = wid * _QPW
        half = _QPW // 2

        pendI = [pltpu.async_copy(
            idxk_hbm.at[k, pl.ds(base_q, _QPW)], idx_v.at[k], semI)
            for k in range(K)]
        chunks = ((a0, a1, a2, a3, semA, 0), (b0, b1, b2, b3, semB, half))
        pend = [[], []]
        for k in range(K):
            pendI[k].wait()
            for ci, (g0, g1, g2, g3, sem, off) in enumerate(chunks):
                pend[ci].append(pltpu.async_copy(
                    valt_hbm.at[idx_v.at[k, pl.ds(off, half)]],
                    (g0, g1, g2, g3)[k], sem))

        for ci, (g0, g1, g2, g3, sem, off) in enumerate(chunks):
            for cp in pend[ci]:
                cp.wait()

            def one_q(q, carry, g0=g0, g1=g1, g2=g2, g3=g3, off=off):
                for c in range(C // 16):
                    sl = pl.ds(c * 16, 16)
                    acc = g0[q, sl] + g1[q, sl]
                    acc = acc + g2[q, sl]
                    acc = acc + g3[q, sl]
                    out_v[off + q, sl] = acc * 0.25
                return carry

            lax.fori_loop(0, half, one_q, 0)
        pltpu.sync_copy(out_v, out_hbm.at[pl.ds(base_q, _QPW)])

    return _sc_gather_mean


_BT = 1024
_NBT = N_TOTAL // _BT


def _tr_body(a_ref, b_ref, y_ref):
    i = pl.program_id(0)
    y_ref[...] = jnp.where(i < _NBT // 2, a_ref[...], b_ref[...]).T


_transpose = pl.pallas_call(
    _tr_body,
    grid=(_NBT,),
    in_specs=[
        pl.BlockSpec((_BT, C), lambda i: (jnp.minimum(i, _NBT // 2 - 1), 0)),
        pl.BlockSpec((_BT, C), lambda i: (jnp.maximum(i - _NBT // 2, 0), 0)),
    ],
    out_specs=pl.BlockSpec((C, _BT), lambda i: (0, i)),
    out_shape=jax.ShapeDtypeStruct((C, N_TOTAL), jnp.float32),
)


def kernel(values, coords, new_coords, shift):
    q_t = jnp.concatenate([coords.T, new_coords.T], axis=1)
    sh = shift.reshape(1, 2)
    sc_gather = _make_sc_gather_mean()
    idxk_a, valt = _topk_a(q_t, coords, sh, values)
    out_a = sc_gather(valt, idxk_a)
    idxk_b = _topk_b(q_t, coords, sh)
    out_b = sc_gather(valt, idxk_b)
    return _transpose(out_a, out_b)


# speedup vs baseline: 1.0192x; 1.0192x over previous
import functools

import jax
import jax.numpy as jnp
from jax import lax
from jax.experimental import pallas as pl
from jax.experimental.pallas import tpu as pltpu
from jax.experimental.pallas import tpu_sc as plsc

N_IN = 2048
N_TOTAL = 8192
C = 128
K = 4


BQ = 1024
N_HALF = N_TOTAL // 2
_NB_HALF = N_HALF // BQ


def _topk_body(qt_ref, cf_ref, sh_ref, idx_ref):
    s = sh_ref[...]
    qx = qt_ref[0:1, :] - s[0:1, 0:1]
    qy = qt_ref[1:2, :] - s[0:1, 1:2]
    cf = cf_ref[...]
    dx = qx - cf[:, 0:1]
    dy = qy - cf[:, 1:2]
    d = dx * dx + dy * dy
    jj = jnp.broadcast_to(
        lax.broadcasted_iota(jnp.int32, (N_IN, 1), 0).astype(jnp.float32),
        (N_IN, BQ))
    rows = []
    for k in range(K):
        m = jnp.min(d, axis=0, keepdims=True)
        cand = jnp.where(d == m, jj, jnp.float32(N_IN))
        amin = jnp.min(cand, axis=0, keepdims=True)
        rows.append(amin)
        if k + 1 < K:
            d = jnp.where(jj == amin, jnp.float32(jnp.inf), d)
    idx_ref[0:K, :] = jnp.concatenate(rows, axis=0).astype(jnp.int32)


def _topk_valt_body(qt_ref, cf_ref, sh_ref, v_ref, idx_ref, valt_ref):
    _topk_body(qt_ref, cf_ref, sh_ref, idx_ref)
    valt_ref[...] = v_ref[...].T


def _make_topk(half, with_valt):
    qt_spec = pl.BlockSpec((2, BQ), lambda i: (0, i + half * _NB_HALF))
    common = [
        pl.BlockSpec((N_IN, 2), lambda i: (0, 0)),
        pl.BlockSpec((1, 2), lambda i: (0, 0)),
    ]
    idx_spec = pl.BlockSpec((2 * K, BQ), lambda i: (0, i))
    idx_shape = jax.ShapeDtypeStruct((2 * K, N_HALF), jnp.int32)
    if with_valt:
        return pl.pallas_call(
            _topk_valt_body,
            grid=(_NB_HALF,),
            in_specs=[qt_spec] + common + [
                pl.BlockSpec((C, N_IN // _NB_HALF), lambda i: (0, i))],
            out_specs=(idx_spec,
                       pl.BlockSpec((N_IN // _NB_HALF, C), lambda i: (i, 0))),
            out_shape=(idx_shape,
                       jax.ShapeDtypeStruct((N_IN, C), jnp.float32)),
        )
    return pl.pallas_call(
        _topk_body,
        grid=(_NB_HALF,),
        in_specs=[qt_spec] + common,
        out_specs=idx_spec,
        out_shape=idx_shape,
    )


_topk_a = _make_topk(0, True)
_topk_b = _make_topk(1, False)


_NC = 2
_NS = 16
_NW = _NC * _NS
_QPW = N_HALF // _NW


@functools.cache
def _make_sc_gather_mean():
    @functools.partial(
        pl.kernel,
        mesh=plsc.VectorSubcoreMesh(core_axis_name="c", subcore_axis_name="s"),
        out_type=jax.ShapeDtypeStruct((N_HALF, C), jnp.float32),
        scratch_types=[
            pltpu.VMEM((K, _QPW), jnp.int32),
            pltpu.VMEM((_QPW // 2, C), jnp.float32),
            pltpu.VMEM((_QPW // 2, C), jnp.float32),
            pltpu.VMEM((_QPW // 2, C), jnp.float32),
            pltpu.VMEM((_QPW // 2, C), jnp.float32),
            pltpu.VMEM((_QPW // 2, C), jnp.float32),
            pltpu.VMEM((_QPW // 2, C), jnp.float32),
            pltpu.VMEM((_QPW // 2, C), jnp.float32),
            pltpu.VMEM((_QPW // 2, C), jnp.float32),
            pltpu.VMEM((_QPW, C), jnp.float32),
            pltpu.SemaphoreType.DMA,
            pltpu.SemaphoreType.DMA,
            pltpu.SemaphoreType.DMA,
        ],
    )
    def _sc_gather_mean(valt_hbm, idxk_hbm, out_hbm, idx_v,
                        a0, a1, a2, a3, b0, b1, b2, b3,
                        out_v, semI, semA, semB):
        wid = lax.axis_index("s") * _NC + lax.axis_index("c")
        base_q = wid * _QPW
        half = _QPW // 2

        pendI = [pltpu.async_copy(
            idxk_hbm.at[k, pl.ds(base_q, _QPW)], idx_v.at[k], semI)
            for k in range(K)]
        chunks = ((a0, a1, a2, a3, semA, 0), (b0, b1, b2, b3, semB, half))
        pend = [[], []]
        for k in range(K):
            pendI[k].wait()
            for ci, (g0, g1, g2, g3, sem, off) in enumerate(chunks):
                pend[ci].append(pltpu.async_copy(
                    valt_hbm.at[idx_v.at[k, pl.ds(off, half)]],
                    (g0, g1, g2, g3)[k], sem))

        for ci, (g0, g1, g2, g3, sem, off) in enumerate(chunks):
            for cp in pend[ci]:
                cp.wait()

            def one_q(q, carry, g0=g0, g1=g1, g2=g2, g3=g3, off=off):
                for c in range(C // 16):
                    sl = pl.ds(c * 16, 16)
                    acc = g0[q, sl] + g1[q, sl]
                    acc = acc + g2[q, sl]
                    acc = acc + g3[q, sl]
                    out_v[off + q, sl] = acc * 0.25
                return carry

            lax.fori_loop(0, half, one_q, 0)
        pltpu.sync_copy(out_v, out_hbm.at[pl.ds(base_q, _QPW)])

    return _sc_gather_mean


_BT = 2048
_NBT = N_TOTAL // _BT


def _tr_body(a_ref, b_ref, y_ref):
    i = pl.program_id(0)
    y_ref[...] = jnp.where(i < _NBT // 2, a_ref[...], b_ref[...]).T


_transpose = pl.pallas_call(
    _tr_body,
    grid=(_NBT,),
    in_specs=[
        pl.BlockSpec((_BT, C), lambda i: (jnp.minimum(i, _NBT // 2 - 1), 0)),
        pl.BlockSpec((_BT, C), lambda i: (jnp.maximum(i - _NBT // 2, 0), 0)),
    ],
    out_specs=pl.BlockSpec((C, _BT), lambda i: (0, i)),
    out_shape=jax.ShapeDtypeStruct((C, N_TOTAL), jnp.float32),
)


def kernel(values, coords, new_coords, shift):
    q_t = jnp.concatenate([coords.T, new_coords.T], axis=1)
    sh = shift.reshape(1, 2)
    sc_gather = _make_sc_gather_mean()
    idxk_a, valt = _topk_a(q_t, coords, sh, values)
    out_a = sc_gather(valt, idxk_a)
    idxk_b = _topk_b(q_t, coords, sh)
    out_b = sc_gather(valt, idxk_b)
    return _transpose(out_a, out_b)


# speedup vs baseline: 1.0241x; 1.0048x over previous
import functools

import jax
import jax.numpy as jnp
from jax import lax
from jax.experimental import pallas as pl
from jax.experimental.pallas import tpu as pltpu
from jax.experimental.pallas import tpu_sc as plsc

N_IN = 2048
N_TOTAL = 8192
C = 128
K = 4


BQ = 1024
N_HALF = N_TOTAL // 2
_NB_HALF = N_HALF // BQ


def _topk_body(qt_ref, cf_ref, sh_ref, idx_ref):
    s = sh_ref[...]
    qx = qt_ref[0:1, :] - s[0:1, 0:1]
    qy = qt_ref[1:2, :] - s[0:1, 1:2]
    cf = cf_ref[...]
    dx = qx - cf[:, 0:1]
    dy = qy - cf[:, 1:2]
    d = dx * dx + dy * dy
    jj = jnp.broadcast_to(
        lax.broadcasted_iota(jnp.int32, (N_IN, 1), 0).astype(jnp.float32),
        (N_IN, BQ))
    rows = []
    for k in range(K):
        m = jnp.min(d, axis=0, keepdims=True)
        cand = jnp.where(d == m, jj, jnp.float32(N_IN))
        amin = jnp.min(cand, axis=0, keepdims=True)
        rows.append(amin)
        if k + 1 < K:
            d = jnp.where(jj == amin, jnp.float32(jnp.inf), d)
    idx_ref[0:K, :] = jnp.concatenate(rows, axis=0).astype(jnp.int32)


def _topk_valt_body(qt_ref, cf_ref, sh_ref, v_ref, idx_ref, valt_ref):
    _topk_body(qt_ref, cf_ref, sh_ref, idx_ref)
    valt_ref[...] = v_ref[...].T


def _make_topk(half, with_valt):
    qt_spec = pl.BlockSpec((2, BQ), lambda i: (0, i + half * _NB_HALF))
    common = [
        pl.BlockSpec((N_IN, 2), lambda i: (0, 0)),
        pl.BlockSpec((1, 2), lambda i: (0, 0)),
    ]
    idx_spec = pl.BlockSpec((2 * K, BQ), lambda i: (0, i))
    idx_shape = jax.ShapeDtypeStruct((2 * K, N_HALF), jnp.int32)
    if with_valt:
        return pl.pallas_call(
            _topk_valt_body,
            grid=(_NB_HALF,),
            in_specs=[qt_spec] + common + [
                pl.BlockSpec((C, N_IN // _NB_HALF), lambda i: (0, i))],
            out_specs=(idx_spec,
                       pl.BlockSpec((N_IN // _NB_HALF, C), lambda i: (i, 0))),
            out_shape=(idx_shape,
                       jax.ShapeDtypeStruct((N_IN, C), jnp.float32)),
        )
    return pl.pallas_call(
        _topk_body,
        grid=(_NB_HALF,),
        in_specs=[qt_spec] + common,
        out_specs=idx_spec,
        out_shape=idx_shape,
    )


_topk_a = _make_topk(0, True)
_topk_b = _make_topk(1, False)


_NC = 2
_NS = 16
_NW = _NC * _NS
_QPW = N_HALF // _NW


@functools.cache
def _make_sc_gather_mean():
    @functools.partial(
        pl.kernel,
        mesh=plsc.VectorSubcoreMesh(core_axis_name="c", subcore_axis_name="s"),
        out_type=jax.ShapeDtypeStruct((N_HALF, C), jnp.float32),
        scratch_types=[
            pltpu.VMEM((K, _QPW), jnp.int32),
            pltpu.VMEM((_QPW, C), jnp.float32),
            pltpu.VMEM((_QPW, C), jnp.float32),
            pltpu.VMEM((_QPW, C), jnp.float32),
            pltpu.VMEM((_QPW, C), jnp.float32),
            pltpu.VMEM((_QPW, C), jnp.float32),
            pltpu.SemaphoreType.DMA,
            pltpu.SemaphoreType.DMA,
        ],
    )
    def _sc_gather_mean(valt_hbm, idxk_hbm, out_hbm, idx_v, g0, g1, g2, g3,
                        out_v, semI, semG):
        wid = lax.axis_index("s") * _NC + lax.axis_index("c")
        base_q = wid * _QPW

        pendI = [pltpu.async_copy(
            idxk_hbm.at[k, pl.ds(base_q, _QPW)], idx_v.at[k], semI)
            for k in range(K)]
        bufs = (g0, g1, g2, g3)
        pendG = []
        for k in range(K):
            pendI[k].wait()
            pendG.append(pltpu.async_copy(
                valt_hbm.at[idx_v.at[k]], bufs[k], semG))
        for cp in pendG:
            cp.wait()

        def one_q(q, carry):
            for c in range(C // 16):
                sl = pl.ds(c * 16, 16)
                acc = g0[q, sl] + g1[q, sl]
                acc = acc + g2[q, sl]
                acc = acc + g3[q, sl]
                out_v[q, sl] = acc * 0.25
            return carry

        lax.fori_loop(0, _QPW, one_q, 0)
        pltpu.sync_copy(out_v, out_hbm.at[pl.ds(base_q, _QPW)])

    return _sc_gather_mean


_BT = 2048
_NBT = N_TOTAL // _BT


def _tr_body(a_ref, b_ref, y_ref):
    i = pl.program_id(0)
    y_ref[...] = jnp.where(i < _NBT // 2, a_ref[...], b_ref[...]).T


_transpose = pl.pallas_call(
    _tr_body,
    grid=(_NBT,),
    in_specs=[
        pl.BlockSpec((_BT, C), lambda i: (jnp.minimum(i, _NBT // 2 - 1), 0)),
        pl.BlockSpec((_BT, C), lambda i: (jnp.maximum(i - _NBT // 2, 0), 0)),
    ],
    out_specs=pl.BlockSpec((C, _BT), lambda i: (0, i)),
    out_shape=jax.ShapeDtypeStruct((C, N_TOTAL), jnp.float32),
)


def kernel(values, coords, new_coords, shift):
    q_t = jnp.concatenate([coords.T, new_coords.T], axis=1)
    sh = shift.reshape(1, 2)
    sc_gather = _make_sc_gather_mean()
    idxk_a, valt = _topk_a(q_t, coords, sh, values)
    out_a = sc_gather(valt, idxk_a)
    idxk_b = _topk_b(q_t, coords, sh)
    out_b = sc_gather(valt, idxk_b)
    return _transpose(out_a, out_b)
